# baseline (device time: 111803 ns/iter reference)
import jax
import jax.numpy as jnp
from jax import lax
from jax.experimental import pallas as pl
from jax.experimental.pallas import tpu as pltpu

N_DEV = 32


def kernel(x, w_mat):
    m_per, k = x.shape
    _, n_per = w_mat.shape

    def body(x_ref, w_ref, out_ref, comm_ref, send_sems, recv_sems):
        my_pos = lax.axis_index("i")
        left = (my_pos - 1) % N_DEV
        right = (my_pos + 1) % N_DEV

        barrier_sem = pltpu.get_barrier_semaphore()
        for nbr in [left, right]:
            pl.semaphore_signal(
                barrier_sem, inc=1,
                device_id=(nbr,), device_id_type=pl.DeviceIdType.MESH,
            )
        pl.semaphore_wait(barrier_sem, 2)

        comm_ref[0] = x_ref[...]
        out_ref[pl.ds(my_pos * m_per, m_per), :] = jnp.dot(
            x_ref[...], w_ref[...], preferred_element_type=jnp.float32
        )

        for h in range(1, N_DEV):
            rdma = pltpu.make_async_remote_copy(
                src_ref=comm_ref.at[h - 1],
                dst_ref=comm_ref.at[h],
                send_sem=send_sems.at[h - 1],
                recv_sem=recv_sems.at[h - 1],
                device_id=(right,),
                device_id_type=pl.DeviceIdType.MESH,
            )
            rdma.start()
            rdma.wait()

            origin = (my_pos - h) % N_DEV
            out_ref[pl.ds(origin * m_per, m_per), :] = jnp.dot(
                comm_ref[h], w_ref[...], preferred_element_type=jnp.float32
            )

    return pl.pallas_call(
        body,
        out_shape=jax.ShapeDtypeStruct((N_DEV * m_per, n_per), jnp.float32),
        in_specs=[
            pl.BlockSpec(memory_space=pltpu.VMEM),
            pl.BlockSpec(memory_space=pltpu.VMEM),
        ],
        out_specs=pl.BlockSpec(memory_space=pltpu.VMEM),
        scratch_shapes=[
            pltpu.VMEM((N_DEV, m_per, k), jnp.float32),
            pltpu.SemaphoreType.DMA((N_DEV - 1,)),
            pltpu.SemaphoreType.DMA((N_DEV - 1,)),
        ],
        compiler_params=pltpu.CompilerParams(collective_id=0),
    )(x, w_mat)


# device time: 96784 ns/iter; 1.1552x vs baseline; 1.1552x over previous
import jax
import jax.numpy as jnp
from jax import lax
from jax.experimental import pallas as pl
from jax.experimental.pallas import tpu as pltpu

N_DEV = 32
R_HOPS = 16
L_HOPS = 15


def kernel(x, w_mat):
    m_per, k = x.shape
    _, n_per = w_mat.shape

    def body(x_ref, w_ref, out_ref,
             comm_r, comm_l, send_r, recv_r, send_l, recv_l):
        my_pos = lax.axis_index("i")
        left = (my_pos - 1) % N_DEV
        right = (my_pos + 1) % N_DEV

        barrier_sem = pltpu.get_barrier_semaphore()
        for nbr in [left, right]:
            pl.semaphore_signal(
                barrier_sem, inc=1,
                device_id=(nbr,), device_id_type=pl.DeviceIdType.MESH,
            )
        pl.semaphore_wait(barrier_sem, 2)

        def fwd(flow_buf, flow_send, flow_recv, hop, dst):
            rdma = pltpu.make_async_remote_copy(
                src_ref=flow_buf.at[hop - 1],
                dst_ref=flow_buf.at[hop],
                send_sem=flow_send.at[hop - 1],
                recv_sem=flow_recv.at[hop - 1],
                device_id=(dst,),
                device_id_type=pl.DeviceIdType.MESH,
            )
            rdma.start()
            return rdma

        comm_r[0] = x_ref[...]
        comm_l[0] = x_ref[...]

        rd_r = fwd(comm_r, send_r, recv_r, 1, right)
        rd_l = fwd(comm_l, send_l, recv_l, 1, left)

        out_ref[pl.ds(my_pos * m_per, m_per), :] = jnp.dot(
            x_ref[...], w_ref[...], preferred_element_type=jnp.float32
        )

        for s in range(1, R_HOPS + 1):
            rd_r.wait_recv()
            if s <= L_HOPS:
                rd_l.wait_recv()
            if s + 1 <= R_HOPS:
                rd_r = fwd(comm_r, send_r, recv_r, s + 1, right)
            if s + 1 <= L_HOPS:
                rd_l = fwd(comm_l, send_l, recv_l, s + 1, left)

            origin_r = (my_pos - s) % N_DEV
            out_ref[pl.ds(origin_r * m_per, m_per), :] = jnp.dot(
                comm_r[s], w_ref[...], preferred_element_type=jnp.float32
            )
            if s <= L_HOPS:
                origin_l = (my_pos + s) % N_DEV
                out_ref[pl.ds(origin_l * m_per, m_per), :] = jnp.dot(
                    comm_l[s], w_ref[...], preferred_element_type=jnp.float32
                )

        for h in range(1, R_HOPS + 1):
            pltpu.make_async_remote_copy(
                src_ref=comm_r.at[h - 1], dst_ref=comm_r.at[h],
                send_sem=send_r.at[h - 1], recv_sem=recv_r.at[h - 1],
                device_id=(right,), device_id_type=pl.DeviceIdType.MESH,
            ).wait_send()
        for h in range(1, L_HOPS + 1):
            pltpu.make_async_remote_copy(
                src_ref=comm_l.at[h - 1], dst_ref=comm_l.at[h],
                send_sem=send_l.at[h - 1], recv_sem=recv_l.at[h - 1],
                device_id=(left,), device_id_type=pl.DeviceIdType.MESH,
            ).wait_send()

    return pl.pallas_call(
        body,
        out_shape=jax.ShapeDtypeStruct((N_DEV * m_per, n_per), jnp.float32),
        in_specs=[
            pl.BlockSpec(memory_space=pltpu.VMEM),
            pl.BlockSpec(memory_space=pltpu.VMEM),
        ],
        out_specs=pl.BlockSpec(memory_space=pltpu.VMEM),
        scratch_shapes=[
            pltpu.VMEM((R_HOPS + 1, m_per, k), jnp.float32),
            pltpu.VMEM((L_HOPS + 1, m_per, k), jnp.float32),
            pltpu.SemaphoreType.DMA((R_HOPS,)),
            pltpu.SemaphoreType.DMA((R_HOPS,)),
            pltpu.SemaphoreType.DMA((L_HOPS,)),
            pltpu.SemaphoreType.DMA((L_HOPS,)),
        ],
        compiler_params=pltpu.CompilerParams(collective_id=0),
    )(x, w_mat)


# device time: 58917 ns/iter; 1.8976x vs baseline; 1.6427x over previous
import jax
import jax.numpy as jnp
import numpy as np
from jax import lax
from jax.experimental import pallas as pl
from jax.experimental.pallas import tpu as pltpu

N_DEV = 32
R_HOPS = 16
L_HOPS = 15

_LOGICAL_COORDS = []
for _z in range(4):
    for _y in range(4):
        _xs = (0, 1) if _y % 2 == 0 else (1, 0)
        for _x in _xs:
            _LOGICAL_COORDS.append((_x, _y, _z))
_COORD_TO_LOGICAL = {c: i for i, c in enumerate(_LOGICAL_COORDS)}

_P = []
for _y in range(4):
    _zs = range(4) if _y % 2 == 0 else range(3, -1, -1)
    for _z in _zs:
        _P.append((_y, _z))
_CYCLE = [(0, y, z) for (y, z) in _P] + [(1, y, z) for (y, z) in reversed(_P)]
assert len(_CYCLE) == N_DEV
for _a, _b in zip(_CYCLE, _CYCLE[1:] + _CYCLE[:1]):
    assert sum(abs(i - j) for i, j in zip(_a, _b)) == 1, (_a, _b)

_HAM = np.array([_COORD_TO_LOGICAL[c] for c in _CYCLE], dtype=np.int32)
_IDX = np.empty(N_DEV, dtype=np.int32)
_IDX[_HAM] = np.arange(N_DEV, dtype=np.int32)


def kernel(x, w_mat):
    m_per, k = x.shape
    _, n_per = w_mat.shape

    def body(x_ref, w_ref, ham_ref, idx_ref, out_ref,
             g_buf, send_r, recv_r, send_l, recv_l):
        my_pos = lax.axis_index("i")
        my_idx = idx_ref[my_pos]
        succ = ham_ref[(my_idx + 1) % N_DEV]
        pred = ham_ref[(my_idx - 1) % N_DEV]

        barrier_sem = pltpu.get_barrier_semaphore()
        for nbr in [pred, succ]:
            pl.semaphore_signal(
                barrier_sem, inc=1,
                device_id=(nbr,), device_id_type=pl.DeviceIdType.MESH,
            )
        pl.semaphore_wait(barrier_sem, 2)

        g_buf[my_pos] = x_ref[...]

        def org_r(h):
            return ham_ref[(my_idx - h) % N_DEV]

        def org_l(h):
            return ham_ref[(my_idx + h) % N_DEV]

        def fwd(h, origin, dst, send_sems, recv_sems):
            rdma = pltpu.make_async_remote_copy(
                src_ref=g_buf.at[origin],
                dst_ref=g_buf.at[origin],
                send_sem=send_sems.at[h - 1],
                recv_sem=recv_sems.at[h - 1],
                device_id=(dst,),
                device_id_type=pl.DeviceIdType.MESH,
            )
            rdma.start()
            return rdma

        rd_r = fwd(1, org_r(0), succ, send_r, recv_r)
        rd_l = fwd(1, org_l(0), pred, send_l, recv_l)

        for s in range(1, R_HOPS + 1):
            rd_r.wait_recv()
            if s <= L_HOPS:
                rd_l.wait_recv()
            if s + 1 <= R_HOPS:
                rd_r = fwd(s + 1, org_r(s), succ, send_r, recv_r)
            if s + 1 <= L_HOPS:
                rd_l = fwd(s + 1, org_l(s), pred, send_l, recv_l)

        out_ref[...] = jnp.dot(
            g_buf[...].reshape(N_DEV * m_per, k), w_ref[...],
            preferred_element_type=jnp.float32,
        )

        for h in range(1, R_HOPS + 1):
            pltpu.make_async_remote_copy(
                src_ref=g_buf.at[0], dst_ref=g_buf.at[0],
                send_sem=send_r.at[h - 1], recv_sem=recv_r.at[h - 1],
                device_id=(succ,), device_id_type=pl.DeviceIdType.MESH,
            ).wait_send()
        for h in range(1, L_HOPS + 1):
            pltpu.make_async_remote_copy(
                src_ref=g_buf.at[0], dst_ref=g_buf.at[0],
                send_sem=send_l.at[h - 1], recv_sem=recv_l.at[h - 1],
                device_id=(pred,), device_id_type=pl.DeviceIdType.MESH,
            ).wait_send()

    return pl.pallas_call(
        body,
        out_shape=jax.ShapeDtypeStruct((N_DEV * m_per, n_per), jnp.float32),
        in_specs=[
            pl.BlockSpec(memory_space=pltpu.VMEM),
            pl.BlockSpec(memory_space=pltpu.VMEM),
            pl.BlockSpec(memory_space=pltpu.SMEM),
            pl.BlockSpec(memory_space=pltpu.SMEM),
        ],
        out_specs=pl.BlockSpec(memory_space=pltpu.VMEM),
        scratch_shapes=[
            pltpu.VMEM((N_DEV, m_per, k), jnp.float32),
            pltpu.SemaphoreType.DMA((R_HOPS,)),
            pltpu.SemaphoreType.DMA((R_HOPS,)),
            pltpu.SemaphoreType.DMA((L_HOPS,)),
            pltpu.SemaphoreType.DMA((L_HOPS,)),
        ],
        compiler_params=pltpu.CompilerParams(collective_id=0),
    )(x, w_mat, jnp.asarray(_HAM), jnp.asarray(_IDX))


# device time: 44605 ns/iter; 2.5065x vs baseline; 1.3209x over previous
import jax
import jax.numpy as jnp
import numpy as np
from jax import lax
from jax.experimental import pallas as pl
from jax.experimental.pallas import tpu as pltpu

N_DEV = 32
R_HOPS = 16
L_HOPS = 15
SUBS = 4

_LOGICAL_COORDS = []
for _z in range(4):
    for _y in range(4):
        _xs = (0, 1) if _y % 2 == 0 else (1, 0)
        for _x in _xs:
            _LOGICAL_COORDS.append((_x, _y, _z))
_COORD_TO_LOGICAL = {c: i for i, c in enumerate(_LOGICAL_COORDS)}

_P = []
for _y in range(4):
    _zs = range(4) if _y % 2 == 0 else range(3, -1, -1)
    for _z in _zs:
        _P.append((_y, _z))
_CYCLE = [(0, y, z) for (y, z) in _P] + [(1, y, z) for (y, z) in reversed(_P)]
assert len(_CYCLE) == N_DEV
for _a, _b in zip(_CYCLE, _CYCLE[1:] + _CYCLE[:1]):
    assert sum(abs(i - j) for i, j in zip(_a, _b)) == 1, (_a, _b)

_HAM = np.array([_COORD_TO_LOGICAL[c] for c in _CYCLE], dtype=np.int32)
_IDX = np.empty(N_DEV, dtype=np.int32)
_IDX[_HAM] = np.arange(N_DEV, dtype=np.int32)


def kernel(x, w_mat):
    m_per, k = x.shape
    _, n_per = w_mat.shape
    sub_m = m_per // SUBS

    def body(x_ref, w_ref, ham_ref, idx_ref, out_ref,
             g_buf, send_r, recv_r, send_l, recv_l):
        my_pos = lax.axis_index("i")
        my_idx = idx_ref[my_pos]
        succ = ham_ref[(my_idx + 1) % N_DEV]
        pred = ham_ref[(my_idx - 1) % N_DEV]

        barrier_sem = pltpu.get_barrier_semaphore()
        for nbr in [pred, succ]:
            pl.semaphore_signal(
                barrier_sem, inc=1,
                device_id=(nbr,), device_id_type=pl.DeviceIdType.MESH,
            )
        pl.semaphore_wait(barrier_sem, 2)

        g_buf[my_pos] = x_ref[...]

        def org_r(h):
            return ham_ref[(my_idx - h) % N_DEV]

        def org_l(h):
            return ham_ref[(my_idx + h) % N_DEV]

        def desc(h, j, origin, dst, send_sems, recv_sems):
            return pltpu.make_async_remote_copy(
                src_ref=g_buf.at[origin, pl.ds(j * sub_m, sub_m)],
                dst_ref=g_buf.at[origin, pl.ds(j * sub_m, sub_m)],
                send_sem=send_sems.at[(h - 1) * SUBS + j],
                recv_sem=recv_sems.at[(h - 1) * SUBS + j],
                device_id=(dst,),
                device_id_type=pl.DeviceIdType.MESH,
            )

        for j in range(SUBS):
            desc(1, j, my_pos, succ, send_r, recv_r).start()
            desc(1, j, my_pos, pred, send_l, recv_l).start()

        for s in range(1, R_HOPS + 1):
            o_r = org_r(s)
            o_l = org_l(s) if s <= L_HOPS else None
            for j in range(SUBS):
                desc(s, j, o_r, succ, send_r, recv_r).wait_recv()
                if s + 1 <= R_HOPS:
                    desc(s + 1, j, o_r, succ, send_r, recv_r).start()
                if o_l is not None:
                    desc(s, j, o_l, pred, send_l, recv_l).wait_recv()
                    if s + 1 <= L_HOPS:
                        desc(s + 1, j, o_l, pred, send_l, recv_l).start()

        out_ref[...] = jnp.dot(
            g_buf[...].reshape(N_DEV * m_per, k), w_ref[...],
            preferred_element_type=jnp.float32,
        )

        for h in range(1, R_HOPS + 1):
            for j in range(SUBS):
                desc(h, j, my_pos, succ, send_r, recv_r).wait_send()
        for h in range(1, L_HOPS + 1):
            for j in range(SUBS):
                desc(h, j, my_pos, pred, send_l, recv_l).wait_send()

    return pl.pallas_call(
        body,
        out_shape=jax.ShapeDtypeStruct((N_DEV * m_per, n_per), jnp.float32),
        in_specs=[
            pl.BlockSpec(memory_space=pltpu.VMEM),
            pl.BlockSpec(memory_space=pltpu.VMEM),
            pl.BlockSpec(memory_space=pltpu.SMEM),
            pl.BlockSpec(memory_space=pltpu.SMEM),
        ],
        out_specs=pl.BlockSpec(memory_space=pltpu.VMEM),
        scratch_shapes=[
            pltpu.VMEM((N_DEV, m_per, k), jnp.float32),
            pltpu.SemaphoreType.DMA((R_HOPS * SUBS,)),
            pltpu.SemaphoreType.DMA((R_HOPS * SUBS,)),
            pltpu.SemaphoreType.DMA((L_HOPS * SUBS,)),
            pltpu.SemaphoreType.DMA((L_HOPS * SUBS,)),
        ],
        compiler_params=pltpu.CompilerParams(collective_id=0),
    )(x, w_mat, jnp.asarray(_HAM), jnp.asarray(_IDX))
